# baseline (speedup 1.0000x reference)
import functools

import jax
import jax.numpy as jnp
from jax import lax
from jax.experimental import pallas as pl
from jax.experimental.pallas import tpu as pltpu
from jax.experimental.pallas import tpu_sc as plsc

B, C, S, H, V = 1024, 26, 50, 32, 1000000
NC, NS, L = 2, 16, 16
NW = NC * NS
BPW = B // NW
R = C * S
CH = 128
RP = 1408
NCHUNK = RP // CH
EPS = 1e-5


def _rsqrt_vec(v):
    i = lax.bitcast_convert_type(v, jnp.int32)
    i = jnp.full((L,), 0x5F3759DF, jnp.int32) - lax.shift_right_logical(i, 1)
    y = lax.bitcast_convert_type(i, jnp.float32)
    for _ in range(3):
        y = y * (1.5 - 0.5 * v * y * y)
    return y


def _bcast_sum(v):
    cs = jnp.cumsum(v)
    sf = lax.rev(jnp.cumsum(lax.rev(v, (0,))), (0,))
    return cs + sf - v


def _body(x_hbm, wt_hbm, pos_hbm, ch_hbm, g_hbm, b_hbm, fw_hbm, out_hbm,
          idx_v, rows_v, pos_v, ch_v, g_v, b_v, fw_v, w_v, pc_v, wb_v,
          out_v, sem):
    wid = lax.axis_index("s") * NC + lax.axis_index("c")

    pltpu.sync_copy(pos_hbm, pos_v)
    pltpu.sync_copy(ch_hbm, ch_v)
    pltpu.sync_copy(g_hbm, g_v)
    pltpu.sync_copy(b_hbm, b_v)
    pltpu.sync_copy(fw_hbm, fw_v)

    iota = lax.iota(jnp.int32, L)

    @pl.loop(0, S)
    def _tables(s):
        v0 = fw_v[s, pl.ds(0, L)]
        v1 = fw_v[s, pl.ds(L, L)]
        m = jnp.maximum(jnp.max(v0), jnp.max(v1))
        e0 = jnp.exp(v0 - m)
        e1 = jnp.exp(v1 - m)
        tot = jnp.broadcast_to(jnp.sum(e0) + jnp.sum(e1), (L,))
        r = 1.0 / tot
        w0 = e0 * r
        w1 = e1 * r
        w_v[s, pl.ds(0, L)] = w0
        w_v[s, pl.ds(L, L)] = w1
        p0 = pos_v[s, pl.ds(0, L)]
        p1 = pos_v[s, pl.ds(L, L)]
        for c in range(C):
            row = c * S + s
            src = w0 if c < L else w1
            t = jnp.where(iota == (c % L), src, 0.0)
            wb_v[row, pl.ds(0, L)] = _bcast_sum(t)
            pc_v[row, pl.ds(0, L)] = ch_v[c, pl.ds(0, L)] + p0
            pc_v[row, pl.ds(L, L)] = ch_v[c, pl.ds(L, L)] + p1

    g0 = g_v[pl.ds(0, L)]
    g1 = g_v[pl.ds(L, L)]
    be0 = b_v[pl.ds(0, L)]
    be1 = b_v[pl.ds(L, L)]

    @pl.loop(0, BPW)
    def _batch(i):
        bidx = wid * BPW + i
        pltpu.sync_copy(x_hbm.at[bidx], idx_v)
        copies = [
            pltpu.async_copy(
                wt_hbm.at[idx_v.at[pl.ds(j * CH, CH)]],
                rows_v.at[pl.ds(j * CH, CH)],
                sem,
            )
            for j in range(NCHUNK)
        ]
        for cp in copies:
            cp.wait()

        @pl.loop(0, S)
        def _step(s):
            acc0 = jnp.zeros((L,), jnp.float32)
            acc1 = jnp.zeros((L,), jnp.float32)
            for c in range(C):
                row = c * S + s
                e0 = rows_v[row, pl.ds(0, L)] + pc_v[row, pl.ds(0, L)]
                e1 = rows_v[row, pl.ds(L, L)] + pc_v[row, pl.ds(L, L)]
                s1 = _bcast_sum(e0 + e1)
                s2 = _bcast_sum(e0 * e0 + e1 * e1)
                mu = s1 * (1.0 / H)
                var = s2 * (1.0 / H) - mu * mu
                rv = _rsqrt_vec(var + EPS)
                a = wb_v[row, pl.ds(0, L)] * rv
                bt = -(mu * a)
                acc0 = acc0 + e0 * a + bt
                acc1 = acc1 + e1 * a + bt
            out_v[s, pl.ds(0, L)] = acc0 * g0 + be0
            out_v[s, pl.ds(L, L)] = acc1 * g1 + be1

        pltpu.sync_copy(out_v, out_hbm.at[bidx])


_sc_call = functools.partial(
    pl.kernel,
    out_type=jax.ShapeDtypeStruct((B, S, H), jnp.float32),
    mesh=plsc.VectorSubcoreMesh(core_axis_name="c", subcore_axis_name="s"),
    compiler_params=pltpu.CompilerParams(
        needs_layout_passes=False, use_tc_tiling_on_sc=False),
    scratch_types=[
        pltpu.VMEM((RP,), jnp.int32),
        pltpu.VMEM((RP, H), jnp.float32),
        pltpu.VMEM((S, H), jnp.float32),
        pltpu.VMEM((C, H), jnp.float32),
        pltpu.VMEM((H,), jnp.float32),
        pltpu.VMEM((H,), jnp.float32),
        pltpu.VMEM((S, 2 * L), jnp.float32),
        pltpu.VMEM((S, 2 * L), jnp.float32),
        pltpu.VMEM((R, H), jnp.float32),
        pltpu.VMEM((R, L), jnp.float32),
        pltpu.VMEM((S, H), jnp.float32),
        pltpu.SemaphoreType.DMA,
    ],
)(_body)


@jax.jit
def kernel(x, word_table, pos_table, ch_table, ln_gamma, ln_beta, fusion_w):
    x2 = jnp.pad(x.reshape(B, R), ((0, 0), (0, RP - R)))
    fw_pad = jnp.full((S, 2 * L), -1e30, jnp.float32)
    fw_pad = fw_pad.at[:, :C].set(fusion_w)
    return _sc_call(x2, word_table, pos_table, ch_table,
                    ln_gamma, ln_beta, fw_pad)


# speedup vs baseline: 1.6535x; 1.6535x over previous
import functools

import jax
import jax.numpy as jnp
from jax import lax
from jax.experimental import pallas as pl
from jax.experimental.pallas import tpu as pltpu
from jax.experimental.pallas import tpu_sc as plsc

B, C, S, H, V = 1024, 26, 50, 32, 1000000
NC, NS, L = 2, 16, 16
NW = NC * NS
BPW = B // NW
R = C * S
RP = 1408
CHUNKS = tuple((j * 104, 104) for j in range(12)) + ((1248, 52),)
EPS = 1e-5


def _rsqrt_vec(v):
    i = lax.bitcast_convert_type(v, jnp.int32)
    i = jnp.full((L,), 0x5F3759DF, jnp.int32) - lax.shift_right_logical(i, 1)
    y = lax.bitcast_convert_type(i, jnp.float32)
    for _ in range(3):
        y = y * (1.5 - 0.5 * v * y * y)
    return y


def _bcast_sum(v):
    cs = jnp.cumsum(v)
    sf = lax.rev(jnp.cumsum(lax.rev(v, (0,))), (0,))
    return cs + sf - v


def _body(x_hbm, wt_hbm, pos_hbm, ch_hbm, g_hbm, b_hbm, fw_hbm, out_hbm,
          idx_v, rows_v, pos_v, ch_v, g_v, b_v, fw_v, w_v, pc_v, wb_v,
          out_v, sem):
    wid = lax.axis_index("s") * NC + lax.axis_index("c")

    pltpu.sync_copy(pos_hbm, pos_v)
    pltpu.sync_copy(ch_hbm, ch_v)
    pltpu.sync_copy(g_hbm, g_v)
    pltpu.sync_copy(b_hbm, b_v)
    pltpu.sync_copy(fw_hbm, fw_v)

    iota = lax.iota(jnp.int32, L)

    @pl.loop(0, S)
    def _tables(s):
        v0 = fw_v[s, pl.ds(0, L)]
        v1 = fw_v[s, pl.ds(L, L)]
        m = jnp.maximum(jnp.max(v0), jnp.max(v1))
        e0 = jnp.exp(v0 - m)
        e1 = jnp.exp(v1 - m)
        tot = jnp.broadcast_to(jnp.sum(e0) + jnp.sum(e1), (L,))
        r = 1.0 / tot
        w0 = e0 * r
        w1 = e1 * r
        w_v[s, pl.ds(0, L)] = w0
        w_v[s, pl.ds(L, L)] = w1
        p0 = pos_v[s, pl.ds(0, L)]
        p1 = pos_v[s, pl.ds(L, L)]
        for c in range(C):
            row = c * S + s
            src = w0 if c < L else w1
            t = jnp.where(iota == (c % L), src, 0.0)
            wb_v[row, pl.ds(0, L)] = _bcast_sum(t)
            pc_v[row, pl.ds(0, L)] = ch_v[c, pl.ds(0, L)] + p0
            pc_v[row, pl.ds(L, L)] = ch_v[c, pl.ds(L, L)] + p1

    g0 = g_v[pl.ds(0, L)]
    g1 = g_v[pl.ds(L, L)]
    be0 = b_v[pl.ds(0, L)]
    be1 = b_v[pl.ds(L, L)]

    @pl.loop(0, BPW)
    def _batch(i):
        bidx = wid * BPW + i
        pltpu.sync_copy(x_hbm.at[bidx], idx_v)
        copies = [
            pltpu.async_copy(
                wt_hbm.at[idx_v.at[pl.ds(off, n)]],
                rows_v.at[pl.ds(off, n)],
                sem,
            )
            for off, n in CHUNKS
        ]
        for cp in copies:
            cp.wait()

        @pl.loop(0, S)
        def _step(s):
            acc0 = jnp.zeros((L,), jnp.float32)
            acc1 = jnp.zeros((L,), jnp.float32)
            for c in range(C):
                row = c * S + s
                e0 = rows_v[row, pl.ds(0, L)] + pc_v[row, pl.ds(0, L)]
                e1 = rows_v[row, pl.ds(L, L)] + pc_v[row, pl.ds(L, L)]
                s1 = _bcast_sum(e0 + e1)
                s2 = _bcast_sum(e0 * e0 + e1 * e1)
                mu = s1 * (1.0 / H)
                var = s2 * (1.0 / H) - mu * mu
                rv = _rsqrt_vec(var + EPS)
                a = wb_v[row, pl.ds(0, L)] * rv
                bt = -(mu * a)
                acc0 = acc0 + e0 * a + bt
                acc1 = acc1 + e1 * a + bt
            out_v[s, pl.ds(0, L)] = acc0 * g0 + be0
            out_v[s, pl.ds(L, L)] = acc1 * g1 + be1

        pltpu.sync_copy(out_v, out_hbm.at[bidx])


_sc_call = functools.partial(
    pl.kernel,
    out_type=jax.ShapeDtypeStruct((B, S, H), jnp.float32),
    mesh=plsc.VectorSubcoreMesh(core_axis_name="c", subcore_axis_name="s"),
    compiler_params=pltpu.CompilerParams(
        needs_layout_passes=False, use_tc_tiling_on_sc=False),
    scratch_types=[
        pltpu.VMEM((RP,), jnp.int32),
        pltpu.VMEM((R, H), jnp.float32),
        pltpu.VMEM((S, H), jnp.float32),
        pltpu.VMEM((C, H), jnp.float32),
        pltpu.VMEM((H,), jnp.float32),
        pltpu.VMEM((H,), jnp.float32),
        pltpu.VMEM((S, 2 * L), jnp.float32),
        pltpu.VMEM((S, 2 * L), jnp.float32),
        pltpu.VMEM((R, H), jnp.float32),
        pltpu.VMEM((R, L), jnp.float32),
        pltpu.VMEM((S, H), jnp.float32),
        pltpu.SemaphoreType.DMA,
    ],
)(_body)


@jax.jit
def kernel(x, word_table, pos_table, ch_table, ln_gamma, ln_beta, fusion_w):
    x2 = jnp.pad(x.reshape(B, R), ((0, 0), (0, RP - R)))
    fw_pad = jnp.full((S, 2 * L), -1e30, jnp.float32)
    fw_pad = fw_pad.at[:, :C].set(fusion_w)
    return _sc_call(x2, word_table, pos_table, ch_table,
                    ln_gamma, ln_beta, fw_pad)


# speedup vs baseline: 1.7809x; 1.0770x over previous
import functools

import jax
import jax.numpy as jnp
from jax import lax
from jax.experimental import pallas as pl
from jax.experimental.pallas import tpu as pltpu
from jax.experimental.pallas import tpu_sc as plsc

B, C, S, H, V = 1024, 26, 50, 32, 1000000
NC, NS, L = 2, 16, 16
NW = NC * NS
BPW = B // NW
R = C * S
RP = 1408
CHUNKS = tuple((j * 104, 104) for j in range(12)) + ((1248, 52),)
EPS = 1e-5


def _rsqrt_vec(v):
    i = lax.bitcast_convert_type(v, jnp.int32)
    i = jnp.full((L,), 0x5F3759DF, jnp.int32) - lax.shift_right_logical(i, 1)
    y = lax.bitcast_convert_type(i, jnp.float32)
    for _ in range(3):
        y = y * (1.5 - 0.5 * v * y * y)
    return y


def _bcast_sum(v):
    cs = jnp.cumsum(v)
    sf = lax.rev(jnp.cumsum(lax.rev(v, (0,))), (0,))
    return cs + sf - v


def _body(x_hbm, wt_hbm, pos_hbm, ch_hbm, g_hbm, b_hbm, fw_hbm, out_hbm,
          idx_v, rows_v, pos_v, ch_v, g_v, b_v, fw_v, w_v, wb_v,
          out_v, sem_g, sem_i):
    wid = lax.axis_index("s") * NC + lax.axis_index("c")
    base = wid * BPW

    pltpu.sync_copy(pos_hbm, pos_v)
    pltpu.sync_copy(ch_hbm, ch_v)
    pltpu.sync_copy(g_hbm, g_v)
    pltpu.sync_copy(b_hbm, b_v)
    pltpu.sync_copy(fw_hbm, fw_v)

    iota = lax.iota(jnp.int32, L)

    @pl.loop(0, S)
    def _tables(s):
        v0 = fw_v[s, pl.ds(0, L)]
        v1 = fw_v[s, pl.ds(L, L)]
        m = jnp.maximum(jnp.max(v0), jnp.max(v1))
        e0 = jnp.exp(v0 - m)
        e1 = jnp.exp(v1 - m)
        tot = jnp.broadcast_to(jnp.sum(e0) + jnp.sum(e1), (L,))
        r = 1.0 / tot
        w0 = e0 * r
        w1 = e1 * r
        w_v[s, pl.ds(0, L)] = w0
        w_v[s, pl.ds(L, L)] = w1
        for c in range(C):
            src = w0 if c < L else w1
            t = jnp.where(iota == (c % L), src, 0.0)
            wb_v[c * S + s, pl.ds(0, L)] = _bcast_sum(t)

    g0 = g_v[pl.ds(0, L)]
    g1 = g_v[pl.ds(L, L)]
    be0 = b_v[pl.ds(0, L)]
    be1 = b_v[pl.ds(L, L)]

    def _fire(buf):
        for off, n in CHUNKS:
            pltpu.async_copy(
                wt_hbm.at[idx_v.at[buf].at[pl.ds(off, n)]],
                rows_v.at[buf].at[pl.ds(off, n)],
                sem_g,
            )

    def _drain(buf):
        pltpu.make_async_copy(
            wt_hbm.at[pl.ds(0, R)], rows_v.at[buf], sem_g).wait()

    def _wait_idx(buf):
        pltpu.make_async_copy(x_hbm.at[0], idx_v.at[buf], sem_i).wait()

    def _compute(buf, bidx):
        @pl.loop(0, S)
        def _step(s):
            p0 = pos_v[s, pl.ds(0, L)]
            p1 = pos_v[s, pl.ds(L, L)]
            acc0 = jnp.zeros((L,), jnp.float32)
            acc1 = jnp.zeros((L,), jnp.float32)
            for c in range(C):
                row = c * S + s
                e0 = (rows_v[buf, row, pl.ds(0, L)]
                      + ch_v[c, pl.ds(0, L)] + p0)
                e1 = (rows_v[buf, row, pl.ds(L, L)]
                      + ch_v[c, pl.ds(L, L)] + p1)
                s1 = _bcast_sum(e0 + e1)
                s2 = _bcast_sum(e0 * e0 + e1 * e1)
                mu = s1 * (1.0 / H)
                var = s2 * (1.0 / H) - mu * mu
                rv = _rsqrt_vec(var + EPS)
                a = wb_v[row, pl.ds(0, L)] * rv
                bt = -(mu * a)
                acc0 = acc0 + e0 * a + bt
                acc1 = acc1 + e1 * a + bt
            out_v[s, pl.ds(0, L)] = acc0 * g0 + be0
            out_v[s, pl.ds(L, L)] = acc1 * g1 + be1

        pltpu.sync_copy(out_v, out_hbm.at[bidx])

    pltpu.sync_copy(x_hbm.at[base], idx_v.at[0])
    _fire(0)
    pltpu.async_copy(x_hbm.at[base + 1], idx_v.at[1], sem_i)

    @pl.loop(0, BPW, step=2)
    def _batch(i):
        b0 = base + i
        _drain(0)
        _wait_idx(1)
        _fire(1)

        @pl.when(i + 2 < BPW)
        def _():
            pltpu.async_copy(x_hbm.at[b0 + 2], idx_v.at[0], sem_i)

        _compute(0, b0)

        _drain(1)

        @pl.when(i + 2 < BPW)
        def _():
            _wait_idx(0)
            _fire(0)

        @pl.when(i + 3 < BPW)
        def _():
            pltpu.async_copy(x_hbm.at[b0 + 3], idx_v.at[1], sem_i)

        _compute(1, b0 + 1)


_sc_call = functools.partial(
    pl.kernel,
    out_type=jax.ShapeDtypeStruct((B, S, H), jnp.float32),
    mesh=plsc.VectorSubcoreMesh(core_axis_name="c", subcore_axis_name="s"),
    compiler_params=pltpu.CompilerParams(
        needs_layout_passes=False, use_tc_tiling_on_sc=False),
    scratch_types=[
        pltpu.VMEM((2, RP), jnp.int32),
        pltpu.VMEM((2, R, H), jnp.float32),
        pltpu.VMEM((S, H), jnp.float32),
        pltpu.VMEM((C, H), jnp.float32),
        pltpu.VMEM((H,), jnp.float32),
        pltpu.VMEM((H,), jnp.float32),
        pltpu.VMEM((S, 2 * L), jnp.float32),
        pltpu.VMEM((S, 2 * L), jnp.float32),
        pltpu.VMEM((R, L), jnp.float32),
        pltpu.VMEM((S, H), jnp.float32),
        pltpu.SemaphoreType.DMA,
        pltpu.SemaphoreType.DMA,
    ],
)(_body)


@jax.jit
def kernel(x, word_table, pos_table, ch_table, ln_gamma, ln_beta, fusion_w):
    x2 = jnp.pad(x.reshape(B, R), ((0, 0), (0, RP - R)))
    fw_pad = jnp.full((S, 2 * L), -1e30, jnp.float32)
    fw_pad = fw_pad.at[:, :C].set(fusion_w)
    return _sc_call(x2, word_table, pos_table, ch_table,
                    ln_gamma, ln_beta, fw_pad)
